# baseline (speedup 1.0000x reference)
import jax
import jax.numpy as jnp
from jax.experimental import pallas as pl
from jax.experimental.pallas import tpu as pltpu
from jax.experimental.pallas import tpu_sc as plsc

K = 8192
D = 256
NPAIR = 16384
SHIFT = 10.0
SCALE2 = 14.426950408889634

BN = 1024
BI = 2048
BJ = 2048
BP = 2048
WG = 128


def _lse_body(cb_ref, a_ref, cbn_ref, cbs_ref, accc_ref, rows_ref):
    @pl.when(pl.program_id(0) == 0)
    def _():
        for r in range(K // BN):
            x = cb_ref[pl.ds(r * BN, BN), :]
            ss = jnp.sum(x * x, axis=1, keepdims=True)
            inv = 1.0 / jnp.maximum(jnp.sqrt(ss), 1e-8)
            y = x * inv
            cbn_ref[pl.ds(r * BN, BN), :] = y.astype(jnp.bfloat16)
            cbs_ref[pl.ds(r * BN, BN), :] = (y * SCALE2).astype(jnp.bfloat16)
        accc_ref[...] = jnp.zeros((1, K), jnp.float32)

    i = pl.program_id(0)
    xi = cbs_ref[pl.ds(i * BI, BI), :]
    rows_ref[...] = jnp.zeros((BI, 1), jnp.float32)

    HB = BJ // 2
    for j in range(K // BJ):
        @pl.when(j >= i)
        def _(j=j):
            for h in range(2):
                cbj = cbn_ref[pl.ds(j * BJ + h * HB, HB), :]
                s2 = jax.lax.dot_general(xi, cbj, (((1,), (1,)), ((), ())),
                                         preferred_element_type=jnp.float32)
                e = jnp.exp2(s2 - SCALE2)
                rows_ref[...] += jnp.sum(e, axis=1, keepdims=True)

                @pl.when(j > i)
                def _(h=h):
                    accc_ref[:, pl.ds(j * BJ + h * HB, HB)] += jnp.sum(
                        e, axis=0, keepdims=True)

    tot = rows_ref[...] + jnp.transpose(accc_ref[:, pl.ds(i * BI, BI)])
    a = SHIFT + jnp.log(tot)
    a_ref[...] = jnp.broadcast_to(a, (BI, 128))


def _lse(codebook):
    return pl.pallas_call(
        _lse_body,
        grid=(K // BI,),
        in_specs=[pl.BlockSpec((K, D), lambda i: (0, 0))],
        out_specs=pl.BlockSpec((BI, 128), lambda i: (i, 0)),
        out_shape=jax.ShapeDtypeStruct((K, 128), jnp.float32),
        scratch_shapes=[pltpu.VMEM((K, D), jnp.bfloat16),
                        pltpu.VMEM((K, D), jnp.bfloat16),
                        pltpu.VMEM((1, K), jnp.float32),
                        pltpu.VMEM((BI, 1), jnp.float32)],
    )(codebook)


def _pair_body(gi_ref, gj_ref, ai_ref, ii_ref, jj_ref, o_ref):
    gi = gi_ref[...]
    gj = gj_ref[...]
    dot = jnp.sum(gi * gj, axis=1, keepdims=True)
    ni = jnp.maximum(jnp.sqrt(jnp.sum(gi * gi, axis=1, keepdims=True)), 1e-8)
    nj = jnp.maximum(jnp.sqrt(jnp.sum(gj * gj, axis=1, keepdims=True)), 1e-8)
    s = dot / (ni * nj)
    x = ai_ref[:, 0:1] - s
    sp = jnp.maximum(x, 0.0) + jnp.log(1.0 + jnp.exp(-jnp.abs(x)))
    loss = jnp.where(ii_ref[...] != jj_ref[...], sp, 0.0)
    part = jnp.sum(loss) / NPAIR

    @pl.when(pl.program_id(0) == 0)
    def _():
        o_ref[0, 0] = 0.0

    o_ref[0, 0] += part


def _pair_loss(gi, gj, ai, ii2, jj2):
    return pl.pallas_call(
        _pair_body,
        grid=(NPAIR // BP,),
        in_specs=[pl.BlockSpec((BP, D), lambda i: (i, 0)),
                  pl.BlockSpec((BP, D), lambda i: (i, 0)),
                  pl.BlockSpec((BP, 128), lambda i: (i, 0)),
                  pl.BlockSpec((BP, 1), lambda i: (i, 0)),
                  pl.BlockSpec((BP, 1), lambda i: (i, 0))],
        out_specs=pl.BlockSpec(memory_space=pltpu.SMEM),
        out_shape=jax.ShapeDtypeStruct((1, 1), jnp.float32),
    )(gi, gj, ai, ii2, jj2)


def _sc_gather(data, idx):
    vdim = data.shape[1]
    mesh = plsc.VectorSubcoreMesh(core_axis_name="core",
                                  subcore_axis_name="subcore")

    @pl.kernel(out_type=jax.ShapeDtypeStruct((NPAIR, vdim), data.dtype),
               mesh=mesh)
    def k(d_hbm, i_hbm, o_hbm):
        def body(i_vmem, o_vmem):
            pltpu.sync_copy(d_hbm.at[i_vmem.at[0]], o_vmem)

        pltpu.emit_pipeline(
            body,
            grid=(NPAIR // WG,),
            in_specs=[pl.BlockSpec((1, WG), lambda i: (0, i))],
            out_specs=[pl.BlockSpec((WG, vdim), lambda i: (i, 0))],
            core_axis_name=("core", "subcore"),
            dimension_semantics=(pltpu.PARALLEL,),
        )(i_hbm, o_hbm)

    return k(data, idx)


def kernel(codebook, indices_pair_list):
    ii = indices_pair_list[:, 0].reshape(1, NPAIR)
    jj = indices_pair_list[:, 1].reshape(1, NPAIR)
    gi = _sc_gather(codebook, ii)
    gj = _sc_gather(codebook, jj)
    a = _lse(codebook)
    ai = _sc_gather(a, ii)
    ii2 = indices_pair_list[:, 0:1]
    jj2 = indices_pair_list[:, 1:2]
    out = _pair_loss(gi, gj, ai, ii2, jj2)
    return out[0, 0]


# speedup vs baseline: 1.0452x; 1.0452x over previous
import jax
import jax.numpy as jnp
from jax.experimental import pallas as pl
from jax.experimental.pallas import tpu as pltpu
from jax.experimental.pallas import tpu_sc as plsc

K = 8192
D = 256
NPAIR = 16384
SHIFT = 10.0
SCALE2 = 14.426950408889634

BN = 1024
BI = 2048
BJ = 2048
BP = 2048
WG = 128


def _lse_body(cb_ref, a_ref, cbn_ref, cbs_ref, accc_ref, rows_ref):
    @pl.when(pl.program_id(0) == 0)
    def _():
        for r in range(K // BN):
            x = cb_ref[pl.ds(r * BN, BN), :]
            ss = jnp.sum(x * x, axis=1, keepdims=True)
            inv = 1.0 / jnp.maximum(jnp.sqrt(ss), 1e-8)
            y = x * inv
            cbn_ref[pl.ds(r * BN, BN), :] = y.astype(jnp.bfloat16)
            cbs_ref[pl.ds(r * BN, BN), :] = (y * SCALE2).astype(jnp.bfloat16)
        accc_ref[...] = jnp.zeros((1, K), jnp.float32)

    i = pl.program_id(0)
    xi = cbs_ref[pl.ds(i * BI, BI), :]
    rows_ref[...] = jnp.zeros((BI, 1), jnp.float32)

    for j in range(K // BJ):
        @pl.when(j >= i)
        def _(j=j):
            cbj = cbn_ref[pl.ds(j * BJ, BJ), :]
            s2 = jax.lax.dot_general(xi, cbj, (((1,), (1,)), ((), ())),
                                     preferred_element_type=jnp.float32)
            e = jnp.exp2(s2 - SCALE2)
            rows_ref[...] += jnp.sum(e, axis=1, keepdims=True)

            @pl.when(j > i)
            def _():
                accc_ref[:, pl.ds(j * BJ, BJ)] += jnp.sum(
                    e, axis=0, keepdims=True)

    tot = rows_ref[...] + jnp.transpose(accc_ref[:, pl.ds(i * BI, BI)])
    a = SHIFT + jnp.log(tot)
    a_ref[...] = jnp.broadcast_to(a, (BI, 128))


def _lse(codebook):
    return pl.pallas_call(
        _lse_body,
        grid=(K // BI,),
        in_specs=[pl.BlockSpec((K, D), lambda i: (0, 0))],
        out_specs=pl.BlockSpec((BI, 128), lambda i: (i, 0)),
        out_shape=jax.ShapeDtypeStruct((K, 128), jnp.float32),
        scratch_shapes=[pltpu.VMEM((K, D), jnp.bfloat16),
                        pltpu.VMEM((K, D), jnp.bfloat16),
                        pltpu.VMEM((1, K), jnp.float32),
                        pltpu.VMEM((BI, 1), jnp.float32)],
    )(codebook)


def _pair_body(gi_ref, gj_ref, ai_ref, ii_ref, jj_ref, o_ref):
    gi = gi_ref[...]
    gj = gj_ref[...]
    dot = jnp.sum(gi * gj, axis=1, keepdims=True)
    ni = jnp.maximum(jnp.sqrt(jnp.sum(gi * gi, axis=1, keepdims=True)), 1e-8)
    nj = jnp.maximum(jnp.sqrt(jnp.sum(gj * gj, axis=1, keepdims=True)), 1e-8)
    s = dot / (ni * nj)
    x = ai_ref[:, 0:1] - s
    sp = jnp.maximum(x, 0.0) + jnp.log(1.0 + jnp.exp(-jnp.abs(x)))
    loss = jnp.where(ii_ref[...] != jj_ref[...], sp, 0.0)
    part = jnp.sum(loss) / NPAIR

    @pl.when(pl.program_id(0) == 0)
    def _():
        o_ref[0, 0] = 0.0

    o_ref[0, 0] += part


def _pair_loss(gi, gj, ai, ii2, jj2):
    return pl.pallas_call(
        _pair_body,
        grid=(NPAIR // BP,),
        in_specs=[pl.BlockSpec((BP, D), lambda i: (i, 0)),
                  pl.BlockSpec((BP, D), lambda i: (i, 0)),
                  pl.BlockSpec((BP, 128), lambda i: (i, 0)),
                  pl.BlockSpec((BP, 1), lambda i: (i, 0)),
                  pl.BlockSpec((BP, 1), lambda i: (i, 0))],
        out_specs=pl.BlockSpec(memory_space=pltpu.SMEM),
        out_shape=jax.ShapeDtypeStruct((1, 1), jnp.float32),
    )(gi, gj, ai, ii2, jj2)


def _sc_gather(data, idx, wg=WG):
    vdim = data.shape[1]
    mesh = plsc.VectorSubcoreMesh(core_axis_name="core",
                                  subcore_axis_name="subcore")

    @pl.kernel(out_type=jax.ShapeDtypeStruct((NPAIR, vdim), data.dtype),
               mesh=mesh)
    def k(d_hbm, i_hbm, o_hbm):
        def body(i_vmem, o_vmem):
            pltpu.sync_copy(d_hbm.at[i_vmem.at[0]], o_vmem)

        pltpu.emit_pipeline(
            body,
            grid=(NPAIR // wg,),
            in_specs=[pl.BlockSpec((1, wg), lambda i: (0, i))],
            out_specs=[pl.BlockSpec((wg, vdim), lambda i: (i, 0))],
            core_axis_name=("core", "subcore"),
            dimension_semantics=(pltpu.PARALLEL,),
        )(i_hbm, o_hbm)

    return k(data, idx)


def kernel(codebook, indices_pair_list):
    ii = indices_pair_list[:, 0].reshape(1, NPAIR)
    jj = indices_pair_list[:, 1].reshape(1, NPAIR)
    gi = _sc_gather(codebook, ii)
    gj = _sc_gather(codebook, jj)
    a = _lse(codebook)
    ai = _sc_gather(a, ii, wg=256)
    ii2 = indices_pair_list[:, 0:1]
    jj2 = indices_pair_list[:, 1:2]
    out = _pair_loss(gi, gj, ai, ii2, jj2)
    return out[0, 0]


# speedup vs baseline: 1.0559x; 1.0103x over previous
import jax
import jax.numpy as jnp
from jax.experimental import pallas as pl
from jax.experimental.pallas import tpu as pltpu
from jax.experimental.pallas import tpu_sc as plsc

K = 8192
D = 256
NPAIR = 16384
SHIFT = 10.0
SCALE2 = 14.426950408889634

BN = 1024
BI = 2048
BJ = 2048
BP = 4096
WG = 128


def _lse_body(cb_ref, a_ref, cbn_ref, cbs_ref, accc_ref, rows_ref):
    @pl.when(pl.program_id(0) == 0)
    def _():
        for r in range(K // BN):
            x = cb_ref[pl.ds(r * BN, BN), :]
            ss = jnp.sum(x * x, axis=1, keepdims=True)
            inv = 1.0 / jnp.maximum(jnp.sqrt(ss), 1e-8)
            y = x * inv
            cbn_ref[pl.ds(r * BN, BN), :] = y.astype(jnp.bfloat16)
            cbs_ref[pl.ds(r * BN, BN), :] = (y * SCALE2).astype(jnp.bfloat16)
        accc_ref[...] = jnp.zeros((1, K), jnp.float32)

    i = pl.program_id(0)
    xi = cbs_ref[pl.ds(i * BI, BI), :]
    rows_ref[...] = jnp.zeros((BI, 1), jnp.float32)

    for j in range(K // BJ):
        @pl.when(j >= i)
        def _(j=j):
            cbj = cbn_ref[pl.ds(j * BJ, BJ), :]
            s2 = jax.lax.dot_general(xi, cbj, (((1,), (1,)), ((), ())),
                                     preferred_element_type=jnp.float32)
            e = jnp.exp2(s2 - SCALE2)
            rows_ref[...] += jnp.sum(e, axis=1, keepdims=True)

            @pl.when(j > i)
            def _():
                accc_ref[:, pl.ds(j * BJ, BJ)] += jnp.sum(
                    e, axis=0, keepdims=True)

    tot = rows_ref[...] + jnp.transpose(accc_ref[:, pl.ds(i * BI, BI)])
    a = SHIFT + jnp.log(tot)
    a_ref[...] = jnp.broadcast_to(a, (BI, 128))


def _lse(codebook):
    return pl.pallas_call(
        _lse_body,
        grid=(K // BI,),
        in_specs=[pl.BlockSpec((K, D), lambda i: (0, 0))],
        out_specs=pl.BlockSpec((BI, 128), lambda i: (i, 0)),
        out_shape=jax.ShapeDtypeStruct((K, 128), jnp.float32),
        scratch_shapes=[pltpu.VMEM((K, D), jnp.bfloat16),
                        pltpu.VMEM((K, D), jnp.bfloat16),
                        pltpu.VMEM((1, K), jnp.float32),
                        pltpu.VMEM((BI, 1), jnp.float32)],
    )(codebook)


def _pair_body(gi_ref, gj_ref, ai_ref, ii_ref, jj_ref, o_ref):
    gi = gi_ref[...]
    gj = gj_ref[...]
    dot = jnp.sum(gi * gj, axis=1, keepdims=True)
    ni = jnp.maximum(jnp.sqrt(jnp.sum(gi * gi, axis=1, keepdims=True)), 1e-8)
    nj = jnp.maximum(jnp.sqrt(jnp.sum(gj * gj, axis=1, keepdims=True)), 1e-8)
    s = dot / (ni * nj)
    x = ai_ref[:, 0:1] - s
    sp = jnp.maximum(x, 0.0) + jnp.log(1.0 + jnp.exp(-jnp.abs(x)))
    loss = jnp.where(ii_ref[...] != jj_ref[...], sp, 0.0)
    part = jnp.sum(loss) / NPAIR

    @pl.when(pl.program_id(0) == 0)
    def _():
        o_ref[0, 0] = 0.0

    o_ref[0, 0] += part


def _pair_loss(gi, gj, ai, ii2, jj2):
    return pl.pallas_call(
        _pair_body,
        grid=(NPAIR // BP,),
        in_specs=[pl.BlockSpec((BP, D), lambda i: (i, 0)),
                  pl.BlockSpec((BP, D), lambda i: (i, 0)),
                  pl.BlockSpec((BP, 128), lambda i: (i, 0)),
                  pl.BlockSpec((BP, 1), lambda i: (i, 0)),
                  pl.BlockSpec((BP, 1), lambda i: (i, 0))],
        out_specs=pl.BlockSpec(memory_space=pltpu.SMEM),
        out_shape=jax.ShapeDtypeStruct((1, 1), jnp.float32),
    )(gi, gj, ai, ii2, jj2)


def _sc_gather(data, idx, wg=WG):
    vdim = data.shape[1]
    mesh = plsc.VectorSubcoreMesh(core_axis_name="core",
                                  subcore_axis_name="subcore")

    @pl.kernel(out_type=jax.ShapeDtypeStruct((NPAIR, vdim), data.dtype),
               mesh=mesh)
    def k(d_hbm, i_hbm, o_hbm):
        def body(i_vmem, o_vmem):
            pltpu.sync_copy(d_hbm.at[i_vmem.at[0]], o_vmem)

        pltpu.emit_pipeline(
            body,
            grid=(NPAIR // wg,),
            in_specs=[pl.BlockSpec((1, wg), lambda i: (0, i))],
            out_specs=[pl.BlockSpec((wg, vdim), lambda i: (i, 0))],
            core_axis_name=("core", "subcore"),
            dimension_semantics=(pltpu.PARALLEL,),
        )(i_hbm, o_hbm)

    return k(data, idx)


def kernel(codebook, indices_pair_list):
    ii = indices_pair_list[:, 0].reshape(1, NPAIR)
    jj = indices_pair_list[:, 1].reshape(1, NPAIR)
    gi = _sc_gather(codebook, ii)
    gj = _sc_gather(codebook, jj)
    a = _lse(codebook)
    ai = _sc_gather(a, ii, wg=256)
    ii2 = indices_pair_list[:, 0:1]
    jj2 = indices_pair_list[:, 1:2]
    out = _pair_loss(gi, gj, ai, ii2, jj2)
    return out[0, 0]
